# Initial kernel scaffold; baseline (speedup 1.0000x reference)
import jax
import jax.numpy as jnp
from jax.experimental import pallas as pl


def kernel(x, pos, edge_index, node2graph, U0, b0, W0, Wb0, U1, b1, W1, Wb1, U2, b2, W2, Wb2):
    raise NotImplementedError("write your pallas kernel here")


# speedup vs baseline: 13.7361x; 13.7361x over previous
import functools

import jax
import jax.numpy as jnp
from jax import lax
from jax.experimental import pallas as pl
from jax.experimental.pallas import tpu as pltpu
from jax.experimental.pallas import tpu_sc as plsc

N = 10000
E = 160000
F = 8
D_IN0 = 128
D_OUT = 16
NG = 128
POS_DIM = 3

CHUNK = 128
NCHUNKS = E // CHUNK
NWORKERS = 32
N_PAD = 10240
ROWS_PER_TILE = N_PAD // 16


def _t0_body(x_ref, w0r_ref, posp_ref, u_ref, hw_ref, posu_ref):
    hw_ref[...] = jnp.dot(x_ref[...], w0r_ref[...],
                          preferred_element_type=jnp.float32)
    posu_ref[...] = jnp.dot(posp_ref[...], u_ref[...],
                            preferred_element_type=jnp.float32)


def _t0(x, w0r, pos_pad, u_cat):
    bn = 1000
    grid = (N // bn,)
    return pl.pallas_call(
        _t0_body,
        grid=grid,
        in_specs=[
            pl.BlockSpec((bn, D_IN0), lambda i: (i, 0)),
            pl.BlockSpec((D_IN0, F * D_OUT), lambda i: (0, 0)),
            pl.BlockSpec((bn, 8), lambda i: (i, 0)),
            pl.BlockSpec((8, 32), lambda i: (0, 0)),
        ],
        out_specs=[
            pl.BlockSpec((bn, F * D_OUT), lambda i: (i, 0)),
            pl.BlockSpec((bn, 32), lambda i: (i, 0)),
        ],
        out_shape=[
            jax.ShapeDtypeStruct((N, F * D_OUT), jnp.float32),
            jax.ShapeDtypeStruct((N, 32), jnp.float32),
        ],
    )(x, w0r, pos_pad, u_cat)


def _tmix_body(part_ref, wb_ref, wnext_ref, hw_ref):
    h = jax.nn.relu(part_ref[0] + part_ref[1] + wb_ref[...])
    hw_ref[...] = jnp.dot(h, wnext_ref[...],
                          preferred_element_type=jnp.float32)


def _tmix(part, wb, wnext):
    bn = 1024
    grid = (N_PAD // bn,)
    return pl.pallas_call(
        _tmix_body,
        grid=grid,
        in_specs=[
            pl.BlockSpec((2, bn, D_OUT), lambda i: (0, i, 0)),
            pl.BlockSpec((1, D_OUT), lambda i: (0, 0)),
            pl.BlockSpec((D_OUT, F * D_OUT), lambda i: (0, 0)),
        ],
        out_specs=pl.BlockSpec((bn, F * D_OUT), lambda i: (i, 0)),
        out_shape=jax.ShapeDtypeStruct((N_PAD, F * D_OUT), jnp.float32),
    )(part, wb, wnext)


def _t2_body(part_ref, wb_ref, n2g_ref, nf_ref, gf_ref):
    h = jax.nn.relu(part_ref[0] + part_ref[1] + wb_ref[...])
    nf_ref[...] = h
    n2g = n2g_ref[0]
    gids = lax.broadcasted_iota(jnp.int32, (NG, n2g.shape[1]), 0)
    onehot = (jnp.broadcast_to(n2g, (NG, n2g.shape[1])) == gids)
    onehot = onehot.astype(jnp.float32)
    gf_part = lax.dot_general(onehot, h, (((1,), (0,)), ((), ())),
                              preferred_element_type=jnp.float32)

    @pl.when(pl.program_id(0) == 0)
    def _():
        gf_ref[...] = jnp.zeros_like(gf_ref)

    gf_ref[...] += gf_part


def _t2(part, wb, n2g3d):
    bn = 1024
    grid = (N_PAD // bn,)
    return pl.pallas_call(
        _t2_body,
        grid=grid,
        in_specs=[
            pl.BlockSpec((2, bn, D_OUT), lambda i: (0, i, 0)),
            pl.BlockSpec((1, D_OUT), lambda i: (0, 0)),
            pl.BlockSpec((1, 1, bn), lambda i: (i, 0, 0)),
        ],
        out_specs=[
            pl.BlockSpec((bn, D_OUT), lambda i: (i, 0)),
            pl.BlockSpec((NG, D_OUT), lambda i: (0, 0)),
        ],
        out_shape=[
            jax.ShapeDtypeStruct((N_PAD, D_OUT), jnp.float32),
            jax.ShapeDtypeStruct((NG, D_OUT), jnp.float32),
        ],
    )(part, wb, n2g3d)


@functools.cache
def _mesh():
    return plsc.VectorSubcoreMesh(core_axis_name="c", subcore_axis_name="s",
                                  num_cores=2, num_subcores=16)


def _sgate_body(posu_hbm, src_hbm, dst_hbm, bias_hbm, gate_hbm,
                sbuf, dbuf, gs, gd, gout, bbuf, sem0, sem1):
    cid = lax.axis_index("c")
    sid = lax.axis_index("s")
    wid = sid * 2 + cid
    pltpu.sync_copy(bias_hbm, bbuf)
    b0 = bbuf[pl.ds(0, 16)]
    b1 = bbuf[pl.ds(16, 16)]
    nchunks = (NCHUNKS - wid + NWORKERS - 1) // NWORKERS

    def chunk_body(i, _):
        off = (wid + i * NWORKERS) * CHUNK
        pltpu.sync_copy(src_hbm.at[pl.ds(off, CHUNK)], sbuf)
        pltpu.sync_copy(dst_hbm.at[pl.ds(off, CHUNK)], dbuf)
        cp0 = pltpu.async_copy(posu_hbm.at[sbuf], gs, sem0)
        cp1 = pltpu.async_copy(posu_hbm.at[dbuf], gd, sem1)
        cp0.wait()
        cp1.wait()

        def edge_body(e, _):
            v0 = jnp.maximum(gs[e, pl.ds(0, 16)] - gd[e, pl.ds(0, 16)] + b0,
                             0.0)
            v1 = jnp.maximum(gs[e, pl.ds(16, 16)] - gd[e, pl.ds(16, 16)] + b1,
                             0.0)
            gout[e, pl.ds(0, 16)] = v0
            gout[e, pl.ds(16, 16)] = v1
            return 0

        lax.fori_loop(0, CHUNK, edge_body, 0)
        pltpu.sync_copy(gout, gate_hbm.at[pl.ds(off, CHUNK), :])
        return 0

    lax.fori_loop(0, nchunks, chunk_body, 0)


def _sgate(posu, src, dst, bias):
    f = pl.kernel(
        _sgate_body,
        out_type=jax.ShapeDtypeStruct((E, 32), jnp.float32),
        mesh=_mesh(),
        compiler_params=pltpu.CompilerParams(use_tc_tiling_on_sc=False),
        scratch_types=[
            pltpu.VMEM((CHUNK,), jnp.int32),
            pltpu.VMEM((CHUNK,), jnp.int32),
            pltpu.VMEM((CHUNK, 32), jnp.float32),
            pltpu.VMEM((CHUNK, 32), jnp.float32),
            pltpu.VMEM((CHUNK, 32), jnp.float32),
            pltpu.VMEM((32,), jnp.float32),
            pltpu.SemaphoreType.DMA,
            pltpu.SemaphoreType.DMA,
        ],
    )
    return f(posu, src, dst, bias)


def _sedge_body(layer, hw_hbm, gate_hbm, src_hbm, dst_hbm, zeros_hbm,
                part_hbm, acc, sbuf, dbuf, gbuf, rows, mbuf, sem0):
    cid = lax.axis_index("c")
    sid = lax.axis_index("s")
    wid = sid * 2 + cid
    pltpu.sync_copy(zeros_hbm.at[pl.ds(sid * ROWS_PER_TILE, ROWS_PER_TILE), :],
                    acc.at[pl.ds(sid * ROWS_PER_TILE, ROWS_PER_TILE), :])
    plsc.subcore_barrier()
    nchunks = (NCHUNKS - wid + NWORKERS - 1) // NWORKERS
    gvec = 16 * (layer // 2)
    glane = 8 * (layer % 2)

    def chunk_body(i, _):
        off = (wid + i * NWORKERS) * CHUNK
        pltpu.sync_copy(src_hbm.at[pl.ds(off, CHUNK)], sbuf)
        pltpu.sync_copy(dst_hbm.at[pl.ds(off, CHUNK)], dbuf)
        pltpu.sync_copy(gate_hbm.at[pl.ds(off, CHUNK), :], gbuf)
        pltpu.async_copy(hw_hbm.at[sbuf], rows, sem0).wait()

        def edge_body(e, _):
            gv = gbuf[e, pl.ds(gvec, 16)]
            acc_v = gv[glane] * rows[e, pl.ds(0, 16)]
            for k in range(1, F):
                acc_v += gv[glane + k] * rows[e, pl.ds(16 * k, 16)]
            mbuf[e, :] = acc_v
            return 0

        lax.fori_loop(0, CHUNK, edge_body, 0)
        pltpu.sync_copy(mbuf, acc.at[dbuf], add=True)
        return 0

    lax.fori_loop(0, nchunks, chunk_body, 0)
    plsc.subcore_barrier()
    pltpu.sync_copy(
        acc.at[pl.ds(sid * ROWS_PER_TILE, ROWS_PER_TILE), :],
        part_hbm.at[cid, pl.ds(sid * ROWS_PER_TILE, ROWS_PER_TILE), :])


def _sedge(layer, hw, gate, src, dst, zeros):
    f = pl.kernel(
        functools.partial(_sedge_body, layer),
        out_type=jax.ShapeDtypeStruct((2, N_PAD, D_OUT), jnp.float32),
        mesh=_mesh(),
        compiler_params=pltpu.CompilerParams(use_tc_tiling_on_sc=False),
        scratch_types=[
            pltpu.VMEM_SHARED((N_PAD, D_OUT), jnp.float32),
            pltpu.VMEM((CHUNK,), jnp.int32),
            pltpu.VMEM((CHUNK,), jnp.int32),
            pltpu.VMEM((CHUNK, 32), jnp.float32),
            pltpu.VMEM((CHUNK, F * D_OUT), jnp.float32),
            pltpu.VMEM((CHUNK, D_OUT), jnp.float32),
            pltpu.SemaphoreType.DMA,
        ],
    )
    return f(hw, gate, src, dst, zeros)


@jax.jit
def kernel(x, pos, edge_index, node2graph,
           U0, b0, W0, Wb0, U1, b1, W1, Wb1, U2, b2, W2, Wb2):
    src = edge_index[0]
    dst = edge_index[1]

    def repack(w, d_in):
        return w.reshape(F, d_in, D_OUT).transpose(1, 0, 2).reshape(d_in, F * D_OUT)

    w0r = repack(W0, D_IN0)
    w1r = repack(W1, D_OUT)
    w2r = repack(W2, D_OUT)

    pos_pad = jnp.pad(pos, ((0, 0), (0, 8 - POS_DIM)))
    u_cat = jnp.zeros((8, 32), jnp.float32)
    u_cat = u_cat.at[:POS_DIM, 0:F].set(U0)
    u_cat = u_cat.at[:POS_DIM, F:2 * F].set(U1)
    u_cat = u_cat.at[:POS_DIM, 2 * F:3 * F].set(U2)
    bias = jnp.concatenate([b0, b1, b2, jnp.zeros((8,), jnp.float32)])

    hw0, posu = _t0(x, w0r, pos_pad, u_cat)
    gate = _sgate(posu, src, dst, bias)

    zeros = jnp.zeros((N_PAD, D_OUT), jnp.float32)
    part0 = _sedge(0, hw0, gate, src, dst, zeros)
    hw1 = _tmix(part0, Wb0.reshape(1, D_OUT), w1r)
    part1 = _sedge(1, hw1, gate, src, dst, zeros)
    hw2 = _tmix(part1, Wb1.reshape(1, D_OUT), w2r)
    part2 = _sedge(2, hw2, gate, src, dst, zeros)

    n2g_pad = jnp.concatenate(
        [node2graph, jnp.full((N_PAD - N,), NG, jnp.int32)])
    n2g3d = n2g_pad.reshape(10, 1, 1024)
    node_feature, graph_feature = _t2(part2, Wb2.reshape(1, D_OUT), n2g3d)
    return graph_feature, node_feature[:N]
